# baseline (device time: 36306 ns/iter reference)
import jax
import jax.numpy as jnp
from jax import lax
from jax.experimental import pallas as pl
from jax.experimental.pallas import tpu as pltpu

N_Y = 4
Q = 4


def kernel(x):
    m_per, n = x.shape
    m2 = m_per // 2
    mq = m2 // Q

    def body(
        x_ref, out_ref, own_buf,
        rbuf, lbuf, rrel, lrel, xrbuf, xlbuf, xrrel, xlrel,
        rd1_ss, rd1_rs, rd2_ss, rd2_rs, rrl_ss, rrl_rs,
        ld1_ss, ld1_rs, ld2_ss, ld2_rs, lrl_ss, lrl_rs,
        x1r_ss, x1r_rs, x2r_ss, x2r_rs, xrr_ss, xrr_rs,
        x1l_ss, x1l_rs, x2l_ss, x2l_rs, xlr_ss, xlr_rs,
        st_sems,
    ):
        my_x = lax.axis_index("x")
        my_y = lax.axis_index("y")
        my_z = lax.axis_index("z")
        Ym = N_Y - 1
        r1 = (my_x, jnp.minimum(my_y + 1, Ym), my_z)
        r2 = (my_x, jnp.minimum(my_y + 2, Ym), my_z)
        l1 = (my_x, jnp.maximum(my_y - 1, 0), my_z)
        l2 = (my_x, jnp.maximum(my_y - 2, 0), my_z)
        peer = (1 - my_x, my_y, my_z)
        edge = jnp.logical_or(my_y == 0, my_y == Ym)
        my_off = my_x * m2
        other_off = (1 - my_x) * m2

        v_rd1_s = my_y <= 2
        v_rd1_r = my_y >= 1
        v_rd2_s = my_y <= 1
        v_rd2_r = my_y >= 2
        v_rrl_s = my_y == 2
        v_rrl_r = my_y == 3
        v_ld1_s = my_y >= 1
        v_ld1_r = my_y <= 2
        v_ld2_s = my_y >= 2
        v_ld2_r = my_y <= 1
        v_lrl_s = my_y == 1
        v_lrl_r = my_y == 0

        bar = pltpu.get_barrier_semaphore()

        def bsig(dev, inc=1):
            pl.semaphore_signal(
                bar, inc=inc, device_id=dev,
                device_id_type=pl.DeviceIdType.MESH,
            )

        @pl.when(v_rd1_s)
        def _():
            bsig(r1)

        @pl.when(v_rd2_s)
        def _():
            bsig(r2)

        @pl.when(v_ld1_s)
        def _():
            bsig(l1)

        @pl.when(v_ld2_s)
        def _():
            bsig(l2)

        pl.semaphore_signal(
            bar, inc=jnp.where(edge, 2, 1), device_id=peer,
            device_id_type=pl.DeviceIdType.MESH,
        )

        own_buf[...] = x_ref[pl.ds(my_off, m2), :]
        own_st = pltpu.make_async_copy(
            x_ref, out_ref.at[pl.ds(my_y * m_per, m_per), :], st_sems.at[0]
        )
        own_st.start()

        pl.semaphore_wait(bar, 4)

        def rcopy(src, dst, ssem, rsem, dev):
            return pltpu.make_async_remote_copy(
                src_ref=src, dst_ref=dst, send_sem=ssem, recv_sem=rsem,
                device_id=dev, device_id_type=pl.DeviceIdType.MESH,
            )

        def pc(ref, q):
            return ref.at[pl.ds(q * mq, mq), :]

        def pc2(ref, i, q):
            return ref.at[i, pl.ds(q * mq, mq), :]

        rd1 = [rcopy(pc(own_buf, q), pc2(rbuf, 0, q),
                     rd1_ss.at[q], rd1_rs.at[q], r1) for q in range(Q)]
        rd2 = [rcopy(pc(own_buf, q), pc2(rbuf, 1, q),
                     rd2_ss.at[q], rd2_rs.at[q], r2) for q in range(Q)]
        rrl = [rcopy(pc2(rbuf, 1, q), pc(rrel, q),
                     rrl_ss.at[q], rrl_rs.at[q], r1) for q in range(Q)]
        ld1 = [rcopy(pc(own_buf, q), pc2(lbuf, 0, q),
                     ld1_ss.at[q], ld1_rs.at[q], l1) for q in range(Q)]
        ld2 = [rcopy(pc(own_buf, q), pc2(lbuf, 1, q),
                     ld2_ss.at[q], ld2_rs.at[q], l2) for q in range(Q)]
        lrl = [rcopy(pc2(lbuf, 1, q), pc(lrel, q),
                     lrl_ss.at[q], lrl_rs.at[q], l1) for q in range(Q)]
        x1r = [rcopy(pc2(rbuf, 0, q), pc2(xrbuf, 0, q),
                     x1r_ss.at[q], x1r_rs.at[q], peer) for q in range(Q)]
        x2r = [rcopy(pc2(rbuf, 1, q), pc2(xrbuf, 1, q),
                     x2r_ss.at[q], x2r_rs.at[q], peer) for q in range(Q)]
        xrr = [rcopy(pc(rrel, q), pc(xrrel, q),
                     xrr_ss.at[q], xrr_rs.at[q], peer) for q in range(Q)]
        x1l = [rcopy(pc2(lbuf, 0, q), pc2(xlbuf, 0, q),
                     x1l_ss.at[q], x1l_rs.at[q], peer) for q in range(Q)]
        x2l = [rcopy(pc2(lbuf, 1, q), pc2(xlbuf, 1, q),
                     x2l_ss.at[q], x2l_rs.at[q], peer) for q in range(Q)]
        xlr = [rcopy(pc(lrel, q), pc(xlrel, q),
                     xlr_ss.at[q], xlr_rs.at[q], peer) for q in range(Q)]

        for q in range(Q):
            @pl.when(v_rd2_s)
            def _(q=q):
                rd2[q].start()

            @pl.when(v_ld2_s)
            def _(q=q):
                ld2[q].start()

        for q in range(Q):
            @pl.when(v_rd1_s)
            def _(q=q):
                rd1[q].start()

            @pl.when(v_ld1_s)
            def _(q=q):
                ld1[q].start()

        for q in range(Q):
            @pl.when(v_rd2_r)
            def _(q=q):
                rd2[q].wait_recv()

            @pl.when(v_rrl_s)
            def _(q=q):
                rrl[q].start()

            @pl.when(v_rd2_r)
            def _(q=q):
                x2r[q].start()

            @pl.when(v_ld2_r)
            def _(q=q):
                ld2[q].wait_recv()

            @pl.when(v_lrl_s)
            def _(q=q):
                lrl[q].start()

            @pl.when(v_ld2_r)
            def _(q=q):
                x2l[q].start()

        for q in range(Q):
            @pl.when(v_rd1_r)
            def _(q=q):
                rd1[q].wait_recv()
                x1r[q].start()

            @pl.when(v_ld1_r)
            def _(q=q):
                ld1[q].wait_recv()
                x1l[q].start()

        for q in range(Q):
            @pl.when(v_rrl_r)
            def _(q=q):
                rrl[q].wait_recv()
                xrr[q].start()

            @pl.when(v_lrl_r)
            def _(q=q):
                lrl[q].wait_recv()
                xlr[q].start()

        c_rd1 = jnp.clip(my_y - 1, 0, Ym)
        c_rd2 = jnp.clip(my_y - 2, 0, Ym)
        c_ld1 = jnp.clip(my_y + 1, 0, Ym)
        c_ld2 = jnp.clip(my_y + 2, 0, Ym)

        def st(i, src, c_off):
            return pltpu.make_async_copy(
                src, out_ref.at[pl.ds(c_off, m2), :], st_sems.at[i]
            )

        stores = [
            (1, v_rd1_r, st(1, rbuf.at[0], c_rd1 * m_per + my_off)),
            (2, v_rd2_r, st(2, rbuf.at[1], c_rd2 * m_per + my_off)),
            (3, v_rrl_r, st(3, rrel, 0 * m_per + my_off)),
            (4, v_ld1_r, st(4, lbuf.at[0], c_ld1 * m_per + my_off)),
            (5, v_ld2_r, st(5, lbuf.at[1], c_ld2 * m_per + my_off)),
            (6, v_lrl_r, st(6, lrel, Ym * m_per + my_off)),
            (7, v_rd1_r, st(7, xrbuf.at[0], c_rd1 * m_per + other_off)),
            (8, v_rd2_r, st(8, xrbuf.at[1], c_rd2 * m_per + other_off)),
            (9, v_rrl_r, st(9, xrrel, 0 * m_per + other_off)),
            (10, v_ld1_r, st(10, xlbuf.at[0], c_ld1 * m_per + other_off)),
            (11, v_ld2_r, st(11, xlbuf.at[1], c_ld2 * m_per + other_off)),
            (12, v_lrl_r, st(12, xlrel, Ym * m_per + other_off)),
        ]
        for i in (1, 2, 3, 4, 5, 6):
            _, v, cp = stores[i - 1]

            @pl.when(v)
            def _(cp=cp):
                cp.start()

        xwaits = [
            (v_rd1_r, x1r, 7), (v_rd2_r, x2r, 8), (v_rrl_r, xrr, 9),
            (v_ld1_r, x1l, 10), (v_ld2_r, x2l, 11), (v_lrl_r, xlr, 12),
        ]
        for v, ds, i in xwaits:
            @pl.when(v)
            def _(ds=ds, i=i):
                for q in range(Q):
                    ds[q].wait_recv()
                stores[i - 1][2].start()

        sends = [
            (v_rd1_s, rd1), (v_rd2_s, rd2), (v_rrl_s, rrl),
            (v_ld1_s, ld1), (v_ld2_s, ld2), (v_lrl_s, lrl),
            (v_rd1_r, x1r), (v_rd2_r, x2r), (v_rrl_r, xrr),
            (v_ld1_r, x1l), (v_ld2_r, x2l), (v_lrl_r, xlr),
        ]
        for v, ds in sends:
            for q in range(Q):
                @pl.when(v)
                def _(ds=ds, q=q):
                    ds[q].wait_send()

        own_st.wait()
        for i, v, cp in stores:
            @pl.when(v)
            def _(cp=cp):
                cp.wait()

    dma = pltpu.SemaphoreType.DMA

    return pl.pallas_call(
        body,
        out_shape=jax.ShapeDtypeStruct((N_Y * m_per, n), x.dtype),
        in_specs=[pl.BlockSpec(memory_space=pltpu.VMEM)],
        out_specs=pl.BlockSpec(memory_space=pltpu.VMEM),
        scratch_shapes=[
            pltpu.VMEM((m2, n), x.dtype),
            pltpu.VMEM((2, m2, n), x.dtype),
            pltpu.VMEM((2, m2, n), x.dtype),
            pltpu.VMEM((m2, n), x.dtype),
            pltpu.VMEM((m2, n), x.dtype),
            pltpu.VMEM((2, m2, n), x.dtype),
            pltpu.VMEM((2, m2, n), x.dtype),
            pltpu.VMEM((m2, n), x.dtype),
            pltpu.VMEM((m2, n), x.dtype),
        ] + [dma((Q,)) for _ in range(24)] + [dma((13,))],
        compiler_params=pltpu.CompilerParams(collective_id=0),
    )(x)


# device time: 29846 ns/iter; 1.2164x vs baseline; 1.2164x over previous
import jax
import jax.numpy as jnp
from jax import lax
from jax.experimental import pallas as pl
from jax.experimental.pallas import tpu as pltpu

N_Y = 4
S = N_Y - 1
Q = 8


def kernel(x):
    m_per, n = x.shape
    m2 = m_per // 2
    mq = m2 // Q

    def body(
        x_ref, out_ref, own_buf, rs_buf, ls_buf, xr_buf, xl_buf,
        rs_ssem, rs_rsem, ls_ssem, ls_rsem,
        xr_ssem, xr_rsem, xl_ssem, xl_rsem,
        st_sems,
    ):
        my_x = lax.axis_index("x")
        my_y = lax.axis_index("y")
        my_z = lax.axis_index("z")
        right = (my_x, jnp.minimum(my_y + 1, N_Y - 1), my_z)
        left = (my_x, jnp.maximum(my_y - 1, 0), my_z)
        peer = (1 - my_x, my_y, my_z)
        has_r = my_y < N_Y - 1
        has_l = my_y > 0
        edge = jnp.logical_or(my_y == 0, my_y == N_Y - 1)
        my_off = my_x * m2
        other_off = (1 - my_x) * m2

        def vs_rs(s):
            return jnp.logical_and(has_r, my_y >= s)

        def vr_rs(s):
            return my_y >= s + 1

        def vs_ls(s):
            return jnp.logical_and(has_l, my_y + s <= N_Y - 1)

        def vr_ls(s):
            return my_y + 1 + s <= N_Y - 1

        bar = pltpu.get_barrier_semaphore()

        @pl.when(has_r)
        def _():
            pl.semaphore_signal(
                bar, inc=1, device_id=right,
                device_id_type=pl.DeviceIdType.MESH,
            )

        @pl.when(has_l)
        def _():
            pl.semaphore_signal(
                bar, inc=1, device_id=left,
                device_id_type=pl.DeviceIdType.MESH,
            )

        pl.semaphore_signal(
            bar, inc=jnp.where(edge, 2, 1), device_id=peer,
            device_id_type=pl.DeviceIdType.MESH,
        )

        own_buf[...] = x_ref[pl.ds(my_off, m2), :]
        own_st = pltpu.make_async_copy(
            x_ref, out_ref.at[pl.ds(my_y * m_per, m_per), :], st_sems.at[0]
        )
        own_st.start()

        pl.semaphore_wait(bar, 3)

        def rcopy(src, dst, ssem, rsem, dev):
            return pltpu.make_async_remote_copy(
                src_ref=src, dst_ref=dst, send_sem=ssem, recv_sem=rsem,
                device_id=dev, device_id_type=pl.DeviceIdType.MESH,
            )

        def sub(ref, q):
            return ref.at[pl.ds(q * mq, mq), :]

        def sub2(ref, s, q):
            return ref.at[s, pl.ds(q * mq, mq), :]

        rs_d = [
            [rcopy(sub(own_buf, q) if s == 0 else sub2(rs_buf, s - 1, q),
                   sub2(rs_buf, s, q),
                   rs_ssem.at[s * Q + q], rs_rsem.at[s * Q + q], right)
             for q in range(Q)]
            for s in range(S)
        ]
        ls_d = [
            [rcopy(sub(own_buf, q) if s == 0 else sub2(ls_buf, s - 1, q),
                   sub2(ls_buf, s, q),
                   ls_ssem.at[s * Q + q], ls_rsem.at[s * Q + q], left)
             for q in range(Q)]
            for s in range(S)
        ]
        xr_d = [
            [rcopy(sub2(rs_buf, s, q), sub2(xr_buf, s, q),
                   xr_ssem.at[s * Q + q], xr_rsem.at[s * Q + q], peer)
             for q in range(Q)]
            for s in range(S)
        ]
        xl_d = [
            [rcopy(sub2(ls_buf, s, q), sub2(xl_buf, s, q),
                   xl_ssem.at[s * Q + q], xl_rsem.at[s * Q + q], peer)
             for q in range(Q)]
            for s in range(S)
        ]

        for q in range(Q):
            @pl.when(vs_rs(0))
            def _(q=q):
                rs_d[0][q].start()

            @pl.when(vs_ls(0))
            def _(q=q):
                ls_d[0][q].start()

        c_rs = [jnp.clip(my_y - 1 - s, 0, N_Y - 1) for s in range(S)]
        c_ls = [jnp.clip(my_y + 1 + s, 0, N_Y - 1) for s in range(S)]
        rs_st = [
            pltpu.make_async_copy(
                rs_buf.at[s],
                out_ref.at[pl.ds(c_rs[s] * m_per + my_off, m2), :],
                st_sems.at[1 + s],
            )
            for s in range(S)
        ]
        ls_st = [
            pltpu.make_async_copy(
                ls_buf.at[s],
                out_ref.at[pl.ds(c_ls[s] * m_per + my_off, m2), :],
                st_sems.at[1 + S + s],
            )
            for s in range(S)
        ]
        xr_st = [
            pltpu.make_async_copy(
                xr_buf.at[s],
                out_ref.at[pl.ds(c_rs[s] * m_per + other_off, m2), :],
                st_sems.at[1 + 2 * S + s],
            )
            for s in range(S)
        ]
        xl_st = [
            pltpu.make_async_copy(
                xl_buf.at[s],
                out_ref.at[pl.ds(c_ls[s] * m_per + other_off, m2), :],
                st_sems.at[1 + 3 * S + s],
            )
            for s in range(S)
        ]

        for s in range(S):
            for q in range(Q):
                @pl.when(vr_rs(s))
                def _(s=s, q=q):
                    rs_d[s][q].wait_recv()

                if s + 1 < S:
                    @pl.when(vs_rs(s + 1))
                    def _(s=s, q=q):
                        rs_d[s + 1][q].start()

                @pl.when(vr_rs(s))
                def _(s=s, q=q):
                    xr_d[s][q].start()

                @pl.when(vr_ls(s))
                def _(s=s, q=q):
                    ls_d[s][q].wait_recv()

                if s + 1 < S:
                    @pl.when(vs_ls(s + 1))
                    def _(s=s, q=q):
                        ls_d[s + 1][q].start()

                @pl.when(vr_ls(s))
                def _(s=s, q=q):
                    xl_d[s][q].start()

            @pl.when(vr_rs(s))
            def _(s=s):
                rs_st[s].start()

            @pl.when(vr_ls(s))
            def _(s=s):
                ls_st[s].start()

        for s in range(S):
            @pl.when(vr_rs(s))
            def _(s=s):
                for q in range(Q):
                    xr_d[s][q].wait_recv()
                xr_st[s].start()

            @pl.when(vr_ls(s))
            def _(s=s):
                for q in range(Q):
                    xl_d[s][q].wait_recv()
                xl_st[s].start()

        for s in range(S):
            for q in range(Q):
                @pl.when(vs_rs(s))
                def _(s=s, q=q):
                    rs_d[s][q].wait_send()

                @pl.when(vs_ls(s))
                def _(s=s, q=q):
                    ls_d[s][q].wait_send()

                @pl.when(vr_rs(s))
                def _(s=s, q=q):
                    xr_d[s][q].wait_send()

                @pl.when(vr_ls(s))
                def _(s=s, q=q):
                    xl_d[s][q].wait_send()

        own_st.wait()
        for s in range(S):
            @pl.when(vr_rs(s))
            def _(s=s):
                rs_st[s].wait()
                xr_st[s].wait()

            @pl.when(vr_ls(s))
            def _(s=s):
                ls_st[s].wait()
                xl_st[s].wait()

    return pl.pallas_call(
        body,
        out_shape=jax.ShapeDtypeStruct((N_Y * m_per, n), x.dtype),
        in_specs=[pl.BlockSpec(memory_space=pltpu.VMEM)],
        out_specs=pl.BlockSpec(memory_space=pltpu.VMEM),
        scratch_shapes=[
            pltpu.VMEM((m2, n), x.dtype),
            pltpu.VMEM((S, m2, n), x.dtype),
            pltpu.VMEM((S, m2, n), x.dtype),
            pltpu.VMEM((S, m2, n), x.dtype),
            pltpu.VMEM((S, m2, n), x.dtype),
            pltpu.SemaphoreType.DMA((S * Q,)),
            pltpu.SemaphoreType.DMA((S * Q,)),
            pltpu.SemaphoreType.DMA((S * Q,)),
            pltpu.SemaphoreType.DMA((S * Q,)),
            pltpu.SemaphoreType.DMA((S * Q,)),
            pltpu.SemaphoreType.DMA((S * Q,)),
            pltpu.SemaphoreType.DMA((S * Q,)),
            pltpu.SemaphoreType.DMA((S * Q,)),
            pltpu.SemaphoreType.DMA((1 + 4 * S,)),
        ],
        compiler_params=pltpu.CompilerParams(collective_id=0),
    )(x)
